# baseline (speedup 1.0000x reference)
import functools

import jax
import jax.numpy as jnp
from jax import lax
from jax.experimental import pallas as pl
from jax.experimental.pallas import tpu as pltpu
from jax.experimental.pallas import tpu_sc as plsc

B = 16384
EMB_DIM = 64
FEAT_DIM = 128

NS = 16
BPW = B // NS
CHUNK = 128
NCH = BPW // CHUNK

_sc_mesh1 = plsc.VectorSubcoreMesh(
    core_axis_name="c", subcore_axis_name="s", num_cores=1, num_subcores=NS
)


@functools.partial(
    pl.kernel,
    out_type=jax.ShapeDtypeStruct((B, EMB_DIM), jnp.float32),
    mesh=_sc_mesh1,
    scratch_types=[
        pltpu.VMEM((BPW,), jnp.int32),
        pltpu.VMEM((BPW, EMB_DIM), jnp.float32),
        pltpu.SemaphoreType.DMA,
    ],
    compiler_params=pltpu.CompilerParams(use_tc_tiling_on_sc=False),
)
def _sc_gather(emb, ids, out, idx_v, rows_v, sem):
    s = lax.axis_index("s")
    base = s * BPW

    pltpu.sync_copy(ids.at[pl.ds(base, BPW)], idx_v)
    copies = []
    for j in range(NCH):
        sl = pl.ds(j * CHUNK, CHUNK)
        copies.append(pltpu.async_copy(emb.at[idx_v.at[sl]], rows_v.at[sl], sem))
    for cp in copies:
        cp.wait()
    pltpu.sync_copy(rows_v, out.at[pl.ds(base, BPW)])


_DN = (((1,), (1,)), ((), ()))


def _tc_body(gu_ref, gi_ref, fu_ref, fi_ref, wu_ref, wi_ref, o_ref):
    o_ref[0] = gu_ref[...] + lax.dot_general(
        fu_ref[...], wu_ref[...], _DN, preferred_element_type=jnp.float32
    )
    o_ref[1] = gi_ref[...] + lax.dot_general(
        fi_ref[...], wi_ref[...], _DN, preferred_element_type=jnp.float32
    )


_BM = 2048


def _tc_call(gu, gi, user_feats, item_feats, wu, wi):
    return pl.pallas_call(
        _tc_body,
        grid=(B // _BM,),
        in_specs=[
            pl.BlockSpec((_BM, EMB_DIM), lambda b: (b, 0)),
            pl.BlockSpec((_BM, EMB_DIM), lambda b: (b, 0)),
            pl.BlockSpec((_BM, FEAT_DIM), lambda b: (b, 0)),
            pl.BlockSpec((_BM, FEAT_DIM), lambda b: (b, 0)),
            pl.BlockSpec((EMB_DIM, FEAT_DIM), lambda b: (0, 0)),
            pl.BlockSpec((EMB_DIM, FEAT_DIM), lambda b: (0, 0)),
        ],
        out_specs=pl.BlockSpec((2, _BM, EMB_DIM), lambda b: (0, b, 0)),
        out_shape=jax.ShapeDtypeStruct((2, B, EMB_DIM), jnp.float32),
    )(gu, gi, user_feats, item_feats, wu, wi)


def kernel(user_ids, item_ids, user_feats, item_feats, user_emb, item_emb,
           user_feat_proj, item_feat_proj):
    gu = _sc_gather(user_emb, user_ids.astype(jnp.int32))
    gi = _sc_gather(item_emb, item_ids.astype(jnp.int32))
    return _tc_call(gu, gi, user_feats, item_feats,
                    user_feat_proj, item_feat_proj)
